# baseline (speedup 1.0000x reference)
import functools

import jax
import jax.numpy as jnp
from jax import lax
from jax.experimental import pallas as pl
from jax.experimental.pallas import tpu as pltpu
from jax.experimental.pallas import tpu_sc as plsc

_DIM = 128
_GAMMA = 12.0
_NW = 32
_C = 80


def _pack_halves(x):
    b = x.astype(jnp.bfloat16)
    lo = lax.bitcast_convert_type(b[:, : _DIM // 2], jnp.uint16)
    hi = lax.bitcast_convert_type(b[:, _DIM // 2:], jnp.uint16)
    w = lo.astype(jnp.uint32) | (hi.astype(jnp.uint32) << 16)
    return lax.bitcast_convert_type(w, jnp.int32)


_S = 10.0


def _pack_i8_nodes(x):
    q = jnp.clip(jnp.round(x * _S), -127, 127).astype(jnp.int8)
    u8 = lax.bitcast_convert_type(q, jnp.uint8).astype(jnp.uint32)
    parts = []
    for j2 in range(2):
        a = u8[:, 32 * j2: 32 * j2 + 16]
        b = u8[:, 32 * j2 + 16: 32 * j2 + 32]
        c = u8[:, 32 * j2 + 64: 32 * j2 + 80]
        d = u8[:, 32 * j2 + 80: 32 * j2 + 96]
        parts.append(a | (b << 8) | (c << 16) | (d << 24))
    return lax.bitcast_convert_type(jnp.concatenate(parts, axis=1), jnp.int32)


def _prep_body(rel_ref, norm_ref, rnw_ref):
    x = norm_ref[...]
    ss = jnp.sum(x * x, axis=-1, keepdims=True)
    inv = lax.rsqrt(jnp.maximum(ss, 1e-24))
    rnw_ref[:, : _DIM // 2] = _pack_halves(rel_ref[...] * _S)
    rnw_ref[:, _DIM // 2:] = _pack_halves(x * inv)


def _prep(rel_table, norm_table):
    n_rel = rel_table.shape[0]
    return pl.pallas_call(
        _prep_body,
        out_shape=jax.ShapeDtypeStruct((n_rel, _DIM), jnp.int32),
    )(rel_table, norm_table)


@functools.cache
def _make_sc_kernel(n_edges):
    epw = n_edges // _NW
    n_chunks = epw // _C

    mesh = plsc.VectorSubcoreMesh(core_axis_name="c", subcore_axis_name="s")

    @functools.partial(
        pl.kernel,
        out_type=jax.ShapeDtypeStruct((_NW, epw), jnp.float32),
        mesh=mesh,
        compiler_params=pltpu.CompilerParams(needs_layout_passes=False,
                                             use_tc_tiling_on_sc=False),
        scratch_types=[
            pltpu.VMEM((4, 3, _C), jnp.int32),
            pltpu.VMEM((4, _C, _DIM // 4), jnp.int32),
            pltpu.VMEM((4, _C, _DIM // 4), jnp.int32),
            pltpu.VMEM((4, _C, _DIM), jnp.int32),
            pltpu.VMEM((epw,), jnp.float32),
            pltpu.SemaphoreType.DMA((4,)),
            pltpu.SemaphoreType.DMA((4,)),
        ],
    )
    def sc_kernel(node_hbm, relnorm_hbm, src_hbm, dst_hbm, eid_hbm, out_hbm,
                  idx4, h2, t2, nr2, o_flat, isems, gsems):
        wid = lax.axis_index("s") * 2 + lax.axis_index("c")
        wbase = wid * epw

        def issue_idx(ci):
            islot = ci % 4
            base = wbase + ci * _C
            pltpu.async_copy(src_hbm.at[pl.ds(base, _C)], idx4.at[islot, 0],
                             isems.at[islot])
            pltpu.async_copy(dst_hbm.at[pl.ds(base, _C)], idx4.at[islot, 1],
                             isems.at[islot])
            pltpu.async_copy(eid_hbm.at[pl.ds(base, _C)], idx4.at[islot, 2],
                             isems.at[islot])

        def wait_idx(ci):
            islot = ci % 4
            base = wbase + ci * _C
            pltpu.make_async_copy(src_hbm.at[pl.ds(base, _C)],
                                  idx4.at[islot, 0], isems.at[islot]).wait()
            pltpu.make_async_copy(dst_hbm.at[pl.ds(base, _C)],
                                  idx4.at[islot, 1], isems.at[islot]).wait()
            pltpu.make_async_copy(eid_hbm.at[pl.ds(base, _C)],
                                  idx4.at[islot, 2], isems.at[islot]).wait()

        def issue(ci, slot):
            islot = ci % 4
            pltpu.async_copy(node_hbm.at[idx4.at[islot, 0]], h2.at[slot],
                             gsems.at[slot])
            pltpu.async_copy(node_hbm.at[idx4.at[islot, 1]], t2.at[slot],
                             gsems.at[slot])
            pltpu.async_copy(relnorm_hbm.at[idx4.at[islot, 2]], nr2.at[slot],
                             gsems.at[slot])

        def wait(ci, slot):
            islot = ci % 4
            pltpu.make_async_copy(node_hbm.at[idx4.at[islot, 0]], h2.at[slot],
                                  gsems.at[slot]).wait()
            pltpu.make_async_copy(node_hbm.at[idx4.at[islot, 1]], t2.at[slot],
                                  gsems.at[slot]).wait()
            pltpu.make_async_copy(relnorm_hbm.at[idx4.at[islot, 2]],
                                  nr2.at[slot], gsems.at[slot]).wait()

        def process(ci, slot):
            h_v, t_v, nr_v = h2.at[slot], t2.at[slot], nr2.at[slot]
            lane = lax.iota(jnp.int32, 16)

            def group(g, carry):
                def edge(k, acc):
                    e = g * 16 + k
                    dj = [None] * 4
                    nj = []
                    for j2 in range(2):
                        h8 = plsc.bitcast(h_v[e, pl.ds(16 * j2, 16)],
                                          jnp.int8)
                        t8 = plsc.bitcast(t_v[e, pl.ds(16 * j2, 16)],
                                          jnp.int8)
                        d8 = h8 - t8
                        da, db = plsc.unpack(
                            d8, format=plsc.PackFormat.INTERLEAVED,
                            preferred_element_type=jnp.int16)
                        dj[2 * j2] = da.astype(jnp.bfloat16)
                        dj[2 * j2 + 1] = db.astype(jnp.bfloat16)
                    accd = jnp.zeros((32,), jnp.bfloat16)
                    for j in range(_DIM // 32):
                        n = plsc.bitcast(
                            nr_v[e, pl.ds(_DIM // 2 + 16 * j, 16)],
                            jnp.bfloat16)
                        nj.append(n)
                        accd = accd + dj[j] * n
                    da, db = plsc.unpack(accd,
                                         format=plsc.PackFormat.INTERLEAVED)
                    dot = jnp.sum(da + db)
                    dotv = jnp.broadcast_to(dot, (16,))
                    dotb = plsc.pack(dotv, dotv,
                                     format=plsc.PackFormat.INTERLEAVED)
                    acca = jnp.zeros((32,), jnp.bfloat16)
                    for j in range(_DIM // 32):
                        r = plsc.bitcast(nr_v[e, pl.ds(16 * j, 16)],
                                         jnp.bfloat16)
                        s = dj[j] + r - dotb * nj[j]
                        acca = acca + jnp.abs(s)
                    aa, ab = plsc.unpack(acca,
                                         format=plsc.PackFormat.INTERLEAVED)
                    res = _GAMMA - jnp.sum(aa + ab) * jnp.float32(1.0 / _S)
                    return jnp.where(lane == k, res, acc)

                acc = lax.fori_loop(0, 16, edge, jnp.zeros((16,), jnp.float32),
                                    unroll=4)
                o_flat[pl.ds(ci * _C + g * 16, 16)] = acc
                return carry

            lax.fori_loop(0, _C // 16, group, 0)

        issue_idx(0)
        issue_idx(1)
        wait_idx(0)
        issue(0, 0)
        issue_idx(2)
        wait_idx(1)
        issue(1, 1)

        def loop_body(ci, carry):
            @pl.when(ci + 3 < n_chunks)
            def _():
                issue_idx(ci + 3)

            @pl.when(ci + 2 < n_chunks)
            def _():
                wait_idx(ci + 2)
                issue(ci + 2, (ci + 2) % 4)

            wait(ci, ci % 4)
            process(ci, ci % 4)
            return carry

        lax.fori_loop(0, n_chunks, loop_body, 0)

        pltpu.sync_copy(o_flat, out_hbm.at[wid])

    return sc_kernel


def kernel(node_emb, rel_table, norm_table, edge_id, src, dst):
    n_edges = edge_id.shape[0]
    relnorm_w = _prep(rel_table.astype(jnp.float32),
                      norm_table.astype(jnp.float32))
    node_w = _pack_i8_nodes(node_emb.astype(jnp.float32))
    out = _make_sc_kernel(n_edges)(node_w, relnorm_w,
                                   src.astype(jnp.int32),
                                   dst.astype(jnp.int32),
                                   edge_id.astype(jnp.int32))
    return out.reshape(n_edges)


# speedup vs baseline: 1.1935x; 1.1935x over previous
import functools

import jax
import jax.numpy as jnp
from jax import lax
from jax.experimental import pallas as pl
from jax.experimental.pallas import tpu as pltpu
from jax.experimental.pallas import tpu_sc as plsc

_DIM = 128
_GAMMA = 12.0
_NW = 32
_C = 80


def _pack_halves(x):
    b = x.astype(jnp.bfloat16)
    lo = lax.bitcast_convert_type(b[:, : _DIM // 2], jnp.uint16)
    hi = lax.bitcast_convert_type(b[:, _DIM // 2:], jnp.uint16)
    w = lo.astype(jnp.uint32) | (hi.astype(jnp.uint32) << 16)
    return lax.bitcast_convert_type(w, jnp.int32)


def _pack_f8_nodes(x):
    f8 = x.astype(jnp.float8_e4m3fn)
    u8 = lax.bitcast_convert_type(f8, jnp.uint8).astype(jnp.uint32)
    parts = []
    for j2 in range(2):
        a = u8[:, 32 * j2: 32 * j2 + 16]
        b = u8[:, 32 * j2 + 16: 32 * j2 + 32]
        c = u8[:, 32 * j2 + 64: 32 * j2 + 80]
        d = u8[:, 32 * j2 + 80: 32 * j2 + 96]
        parts.append(a | (b << 8) | (c << 16) | (d << 24))
    return lax.bitcast_convert_type(jnp.concatenate(parts, axis=1), jnp.int32)


def _prep_body(rel_ref, norm_ref, rnw_ref):
    x = norm_ref[...]
    ss = jnp.sum(x * x, axis=-1, keepdims=True)
    inv = lax.rsqrt(jnp.maximum(ss, 1e-24))
    rnw_ref[:, : _DIM // 2] = _pack_halves(rel_ref[...])
    rnw_ref[:, _DIM // 2:] = _pack_halves(x * inv)


def _prep(rel_table, norm_table):
    n_rel = rel_table.shape[0]
    return pl.pallas_call(
        _prep_body,
        out_shape=jax.ShapeDtypeStruct((n_rel, _DIM), jnp.int32),
    )(rel_table, norm_table)


@functools.cache
def _make_sc_kernel(n_edges):
    epw = n_edges // _NW
    n_chunks = epw // _C

    mesh = plsc.VectorSubcoreMesh(core_axis_name="c", subcore_axis_name="s")

    @functools.partial(
        pl.kernel,
        out_type=jax.ShapeDtypeStruct((_NW, epw), jnp.float32),
        mesh=mesh,
        compiler_params=pltpu.CompilerParams(needs_layout_passes=False,
                                             use_tc_tiling_on_sc=False),
        scratch_types=[
            pltpu.VMEM((4, 3, _C), jnp.int32),
            pltpu.VMEM((4, _C, _DIM // 4), jnp.int32),
            pltpu.VMEM((4, _C, _DIM // 4), jnp.int32),
            pltpu.VMEM((4, _C, _DIM), jnp.int32),
            pltpu.VMEM((epw,), jnp.float32),
            pltpu.SemaphoreType.DMA((4,)),
            pltpu.SemaphoreType.DMA((4,)),
        ],
    )
    def sc_kernel(node_hbm, relnorm_hbm, src_hbm, dst_hbm, eid_hbm, out_hbm,
                  idx4, h2, t2, nr2, o_flat, isems, gsems):
        wid = lax.axis_index("s") * 2 + lax.axis_index("c")
        wbase = wid * epw

        def issue_idx(ci):
            islot = ci % 4
            base = wbase + ci * _C
            pltpu.async_copy(src_hbm.at[pl.ds(base, _C)], idx4.at[islot, 0],
                             isems.at[islot])
            pltpu.async_copy(dst_hbm.at[pl.ds(base, _C)], idx4.at[islot, 1],
                             isems.at[islot])
            pltpu.async_copy(eid_hbm.at[pl.ds(base, _C)], idx4.at[islot, 2],
                             isems.at[islot])

        def wait_idx(ci):
            islot = ci % 4
            base = wbase + ci * _C
            pltpu.make_async_copy(src_hbm.at[pl.ds(base, _C)],
                                  idx4.at[islot, 0], isems.at[islot]).wait()
            pltpu.make_async_copy(dst_hbm.at[pl.ds(base, _C)],
                                  idx4.at[islot, 1], isems.at[islot]).wait()
            pltpu.make_async_copy(eid_hbm.at[pl.ds(base, _C)],
                                  idx4.at[islot, 2], isems.at[islot]).wait()

        def issue(ci, slot):
            islot = ci % 4
            pltpu.async_copy(node_hbm.at[idx4.at[islot, 0]], h2.at[slot],
                             gsems.at[slot])
            pltpu.async_copy(node_hbm.at[idx4.at[islot, 1]], t2.at[slot],
                             gsems.at[slot])
            pltpu.async_copy(relnorm_hbm.at[idx4.at[islot, 2]], nr2.at[slot],
                             gsems.at[slot])

        def wait(ci, slot):
            islot = ci % 4
            pltpu.make_async_copy(node_hbm.at[idx4.at[islot, 0]], h2.at[slot],
                                  gsems.at[slot]).wait()
            pltpu.make_async_copy(node_hbm.at[idx4.at[islot, 1]], t2.at[slot],
                                  gsems.at[slot]).wait()
            pltpu.make_async_copy(relnorm_hbm.at[idx4.at[islot, 2]],
                                  nr2.at[slot], gsems.at[slot]).wait()

        def process(ci, slot):
            h_v, t_v, nr_v = h2.at[slot], t2.at[slot], nr2.at[slot]
            lane = lax.iota(jnp.int32, 16)

            def group(g, carry):
                def edge(k, acc):
                    e = g * 16 + k
                    dj = [None] * 4
                    nj = []
                    for j2 in range(2):
                        xh = plsc.bitcast(h_v[e, pl.ds(16 * j2, 16)],
                                          jnp.float8_e4m3fn)
                        xt = plsc.bitcast(t_v[e, pl.ds(16 * j2, 16)],
                                          jnp.float8_e4m3fn)
                        ha, hb = plsc.unpack(
                            xh, format=plsc.PackFormat.INTERLEAVED,
                            preferred_element_type=jnp.bfloat16)
                        ta, tb = plsc.unpack(
                            xt, format=plsc.PackFormat.INTERLEAVED,
                            preferred_element_type=jnp.bfloat16)
                        dj[2 * j2] = ha - ta
                        dj[2 * j2 + 1] = hb - tb
                    accd = jnp.zeros((32,), jnp.bfloat16)
                    for j in range(_DIM // 32):
                        n = plsc.bitcast(
                            nr_v[e, pl.ds(_DIM // 2 + 16 * j, 16)],
                            jnp.bfloat16)
                        nj.append(n)
                        accd = accd + dj[j] * n
                    da, db = plsc.unpack(accd,
                                         format=plsc.PackFormat.INTERLEAVED)
                    dot = jnp.sum(da + db)
                    dotv = jnp.broadcast_to(dot, (16,))
                    dotb = plsc.pack(dotv, dotv,
                                     format=plsc.PackFormat.INTERLEAVED)
                    acca = jnp.zeros((32,), jnp.bfloat16)
                    for j in range(_DIM // 32):
                        r = plsc.bitcast(nr_v[e, pl.ds(16 * j, 16)],
                                         jnp.bfloat16)
                        s = dj[j] + r - dotb * nj[j]
                        acca = acca + jnp.abs(s)
                    aa, ab = plsc.unpack(acca,
                                         format=plsc.PackFormat.INTERLEAVED)
                    res = _GAMMA - jnp.sum(aa + ab)
                    return jnp.where(lane == k, res, acc)

                acc = lax.fori_loop(0, 16, edge, jnp.zeros((16,), jnp.float32),
                                    unroll=4)
                o_flat[pl.ds(ci * _C + g * 16, 16)] = acc
                return carry

            lax.fori_loop(0, _C // 16, group, 0)

        issue_idx(0)
        issue_idx(1)
        wait_idx(0)
        issue(0, 0)
        issue_idx(2)
        wait_idx(1)
        issue(1, 1)

        def loop_body(ci, carry):
            @pl.when(ci + 3 < n_chunks)
            def _():
                issue_idx(ci + 3)

            @pl.when(ci + 2 < n_chunks)
            def _():
                wait_idx(ci + 2)
                issue(ci + 2, (ci + 2) % 4)

            wait(ci, ci % 4)
            process(ci, ci % 4)
            return carry

        lax.fori_loop(0, n_chunks, loop_body, 0)

        pltpu.sync_copy(o_flat, out_hbm.at[wid])

    return sc_kernel


def kernel(node_emb, rel_table, norm_table, edge_id, src, dst):
    n_edges = edge_id.shape[0]
    relnorm_w = _prep(rel_table.astype(jnp.float32),
                      norm_table.astype(jnp.float32))
    node_w = _pack_f8_nodes(node_emb.astype(jnp.float32))
    out = _make_sc_kernel(n_edges)(node_w, relnorm_w,
                                   src.astype(jnp.int32),
                                   dst.astype(jnp.int32),
                                   edge_id.astype(jnp.int32))
    return out.reshape(n_edges)
